# baseline (device time: 34607 ns/iter reference)
import os

import jax
import jax.numpy as jnp
from jax import lax
from jax.experimental import pallas as pl
from jax.experimental.pallas import tpu as pltpu

N_DEV = 8
_INTERPRET = os.environ.get("KERNEL_INTERPRET", "0") == "1"
_MM_DTYPE = jnp.bfloat16 if os.environ.get("KERNEL_MM_BF16") == "1" else jnp.float8_e4m3fn


def _mm(a, b):
    return lax.dot_general(
        a, b, (((1,), (0,)), ((), ())), preferred_element_type=jnp.float32
    )


def kernel(x, w_mat, scale_x, scale_w):
    m_glob, k_loc = x.shape
    k_glob, n_out = w_mat.shape
    mb = m_glob // N_DEV
    half = (N_DEV // 2) * k_loc

    def body(x_ref, w_hbm, sx_ref, sw_ref, out_ref,
             x8_ref, xg_ref, wf_ref, wg_ref, send_sems, recv_sems, wdma_sems):
        me = lax.axis_index("i")

        def kblk(p):
            return (me - p) % N_DEV

        def start_wdma(step, slot):
            cp = pltpu.make_async_copy(
                w_hbm.at[pl.ds(kblk(step) * k_loc, k_loc), :],
                wf_ref.at[slot],
                wdma_sems.at[slot],
            )
            cp.start()
            return cp

        wcp = [start_wdma(0, 0)]
        x8_ref[...] = x_ref[...].astype(jnp.float8_e4m3fn)
        xg_ref[:, 0:k_loc] = x8_ref[pl.ds(me * mb, mb), :]

        barrier = pltpu.get_barrier_semaphore()
        for d in range(1, N_DEV):
            pl.semaphore_signal(
                barrier, inc=1,
                device_id=((me + d) % N_DEV,),
                device_id_type=pl.DeviceIdType.MESH,
            )
        pl.semaphore_wait(barrier, N_DEV - 1)

        rdmas = []
        for d in range(1, N_DEV):
            dst = (me + d) % N_DEV
            rdma = pltpu.make_async_remote_copy(
                src_ref=x8_ref.at[pl.ds(dst * mb, mb), :],
                dst_ref=xg_ref.at[:, pl.ds(d * k_loc, k_loc)],
                send_sem=send_sems.at[d - 1],
                recv_sem=recv_sems.at[d - 1],
                device_id=(dst,),
                device_id_type=pl.DeviceIdType.MESH,
            )
            rdma.start()
            rdmas.append(rdma)

        wcp.append(start_wdma(1, 1))

        for p in range(4):
            wcp[p].wait()
            wg_ref[p * k_loc:(p + 1) * k_loc, :] = wf_ref[p % 2].astype(_MM_DTYPE)
            if p + 2 < N_DEV:
                wcp.append(start_wdma(p + 2, p % 2))
        for d in range(1, 4):
            rdmas[d - 1].wait_recv()
        out_ref[...] = _mm(xg_ref[:, 0:half], wg_ref[0:half, :])

        for p in range(4, N_DEV):
            wcp[p].wait()
            wg_ref[p * k_loc:(p + 1) * k_loc, :] = wf_ref[p % 2].astype(_MM_DTYPE)
            if p + 2 < N_DEV:
                wcp.append(start_wdma(p + 2, p % 2))
        for d in range(4, N_DEV):
            rdmas[d - 1].wait_recv()
        acc = out_ref[...] + _mm(xg_ref[:, half:], wg_ref[half:, :])

        y = acc * (sx_ref[0] * sw_ref[0])
        out_ref[...] = y * jax.nn.sigmoid(jnp.clip(y, -60.0, 60.0))

        for d in range(1, N_DEV):
            rdmas[d - 1].wait_send()

    return pl.pallas_call(
        body,
        out_shape=jax.ShapeDtypeStruct((mb, n_out), jnp.float32),
        in_specs=[
            pl.BlockSpec(memory_space=pltpu.VMEM),
            pl.BlockSpec(memory_space=pltpu.MemorySpace.HBM),
            pl.BlockSpec(memory_space=pltpu.SMEM),
            pl.BlockSpec(memory_space=pltpu.SMEM),
        ],
        out_specs=pl.BlockSpec(memory_space=pltpu.VMEM),
        scratch_shapes=[
            pltpu.VMEM((m_glob, k_loc), jnp.float8_e4m3fn),
            pltpu.VMEM((mb, k_glob), jnp.float8_e4m3fn),
            pltpu.VMEM((2, k_loc, n_out), jnp.float32),
            pltpu.VMEM((k_glob, n_out), _MM_DTYPE),
            pltpu.SemaphoreType.DMA((N_DEV - 1,)),
            pltpu.SemaphoreType.DMA((N_DEV - 1,)),
            pltpu.SemaphoreType.DMA((2,)),
        ],
        compiler_params=pltpu.CompilerParams(
            collective_id=0, vmem_limit_bytes=100 * 1024 * 1024
        ),
        interpret=pltpu.InterpretParams() if _INTERPRET else False,
    )(x, w_mat, scale_x, scale_w)


# device time: 32476 ns/iter; 1.0656x vs baseline; 1.0656x over previous
import os

import jax
import jax.numpy as jnp
from jax import lax
from jax.experimental import pallas as pl
from jax.experimental.pallas import tpu as pltpu

N_DEV = 8
_INTERPRET = os.environ.get("KERNEL_INTERPRET", "0") == "1"
_MM_DTYPE = jnp.bfloat16 if os.environ.get("KERNEL_MM_BF16") == "1" else jnp.float8_e4m3fn
_ABLATE = os.environ.get("KERNEL_ABLATE", "")


def _mm(a, b):
    return lax.dot_general(
        a, b, (((1,), (0,)), ((), ())), preferred_element_type=jnp.float32
    )


def kernel(x, w_mat, scale_x, scale_w):
    m_glob, k_loc = x.shape
    k_glob, n_out = w_mat.shape
    mb = m_glob // N_DEV
    n_chunk = 4
    spc = N_DEV // n_chunk

    def body(x_ref, w_hbm, sx_ref, sw_ref, out_hbm,
             x8_ref, xg_ref, wf_ref, wg_ref, acc_ref,
             send_sems, recv_sems, wdma_sems, odma_sem):
        me = lax.axis_index("i")

        do_comm = _ABLATE != "nocomm"
        do_w = _ABLATE not in ("nowstream", "t1", "t3")
        do_gemm = _ABLATE not in ("nogemm", "t1", "t3")
        link_test = _ABLATE in ("t1", "t3")

        def kblk(p):
            return (me - p) % N_DEV

        def start_wdma(step, slot):
            cp = pltpu.make_async_copy(
                w_hbm.at[pl.ds(kblk(step) * k_loc, k_loc), :],
                wf_ref.at[slot],
                wdma_sems.at[slot],
            )
            cp.start()
            return cp

        wcp = [start_wdma(0, 0) if do_w else None]
        x8_ref[...] = x_ref[...].astype(jnp.float8_e4m3fn)
        xg_ref[:, 0:k_loc] = x8_ref[pl.ds(me * mb, mb), :]

        if do_comm:
            barrier = pltpu.get_barrier_semaphore()
            for d in range(1, N_DEV):
                pl.semaphore_signal(
                    barrier, inc=1,
                    device_id=((me + d) % N_DEV,),
                    device_id_type=pl.DeviceIdType.MESH,
                )
            pl.semaphore_wait(barrier, N_DEV - 1)

        if link_test:
            xn = me + 1 - 2 * (me % 2)
            yn = me + 3 - 2 * (me % 4)
            zn = (me + 4) % N_DEV
            nbrs = [zn] if _ABLATE == "t1" else [xn, yn, zn]
            trdmas = []
            for i, nb in enumerate(nbrs):
                r = pltpu.make_async_remote_copy(
                    src_ref=x8_ref.at[pl.ds(i * mb, mb), :],
                    dst_ref=xg_ref.at[:, pl.ds((i + 1) * k_loc, k_loc)],
                    send_sem=send_sems.at[i],
                    recv_sem=recv_sems.at[i],
                    device_id=(nb,),
                    device_id_type=pl.DeviceIdType.MESH,
                )
                r.start()
                trdmas.append(r)
            for r in trdmas:
                r.wait_recv()
            for r in trdmas:
                r.wait_send()

        rdmas = []
        if do_comm and not link_test:
            for d in range(1, N_DEV):
                dst = (me + d) % N_DEV
                rdma = pltpu.make_async_remote_copy(
                    src_ref=x8_ref.at[pl.ds(dst * mb, mb), :],
                    dst_ref=xg_ref.at[:, pl.ds(d * k_loc, k_loc)],
                    send_sem=send_sems.at[d - 1],
                    recv_sem=recv_sems.at[d - 1],
                    device_id=(dst,),
                    device_id_type=pl.DeviceIdType.MESH,
                )
                rdma.start()
                rdmas.append(rdma)

        if do_w:
            wcp.append(start_wdma(1, 1))

        for c in range(n_chunk):
            lo, hi = c * spc, (c + 1) * spc
            for p in range(lo, hi):
                if do_w:
                    wcp[p].wait()
                    wg_ref[p * k_loc:(p + 1) * k_loc, :] = (
                        wf_ref[p % 2].astype(_MM_DTYPE)
                    )
                    if p + 2 < N_DEV:
                        wcp.append(start_wdma(p + 2, p % 2))
            if do_comm and not link_test:
                for d in range(max(lo, 1), hi):
                    rdmas[d - 1].wait_recv()
            if do_gemm:
                g = _mm(
                    xg_ref[:, lo * k_loc:hi * k_loc],
                    wg_ref[lo * k_loc:hi * k_loc, :],
                )
                if c == 0:
                    acc_ref[...] = g
                elif c < n_chunk - 1:
                    acc_ref[...] += g
                else:
                    y = (acc_ref[...] + g) * (sx_ref[0] * sw_ref[0])
                    acc_ref[...] = y * jax.nn.sigmoid(jnp.clip(y, -60.0, 60.0))
        if not do_gemm:
            acc_ref[...] = jnp.zeros((mb, n_out), jnp.float32)

        ocp = pltpu.make_async_copy(acc_ref, out_hbm, odma_sem)
        ocp.start()
        if do_comm and not link_test:
            for d in range(1, N_DEV):
                rdmas[d - 1].wait_send()
        ocp.wait()

    return pl.pallas_call(
        body,
        out_shape=jax.ShapeDtypeStruct((mb, n_out), jnp.float32),
        in_specs=[
            pl.BlockSpec(memory_space=pltpu.VMEM),
            pl.BlockSpec(memory_space=pltpu.MemorySpace.HBM),
            pl.BlockSpec(memory_space=pltpu.SMEM),
            pl.BlockSpec(memory_space=pltpu.SMEM),
        ],
        out_specs=pl.BlockSpec(memory_space=pltpu.MemorySpace.HBM),
        scratch_shapes=[
            pltpu.VMEM((m_glob, k_loc), jnp.float8_e4m3fn),
            pltpu.VMEM((mb, k_glob), jnp.float8_e4m3fn),
            pltpu.VMEM((2, k_loc, n_out), jnp.float32),
            pltpu.VMEM((k_glob, n_out), _MM_DTYPE),
            pltpu.VMEM((mb, n_out), jnp.float32),
            pltpu.SemaphoreType.DMA((N_DEV - 1,)),
            pltpu.SemaphoreType.DMA((N_DEV - 1,)),
            pltpu.SemaphoreType.DMA((2,)),
            pltpu.SemaphoreType.DMA,
        ],
        compiler_params=pltpu.CompilerParams(
            collective_id=None if _ABLATE == "nocomm" else 0,
            vmem_limit_bytes=100 * 1024 * 1024,
        ),
        interpret=pltpu.InterpretParams() if _INTERPRET else False,
    )(x, w_mat, scale_x, scale_w)


# device time: 29704 ns/iter; 1.1651x vs baseline; 1.0933x over previous
import os

import jax
import jax.numpy as jnp
from jax import lax
from jax.experimental import pallas as pl
from jax.experimental.pallas import tpu as pltpu

N_DEV = 8
_INTERPRET = os.environ.get("KERNEL_INTERPRET", "0") == "1"
_MM_DTYPE = jnp.bfloat16 if os.environ.get("KERNEL_MM_BF16") == "1" else jnp.float8_e4m3fn
_ABLATE = os.environ.get("KERNEL_ABLATE", "")


def _mm(a, b):
    return lax.dot_general(
        a, b, (((1,), (0,)), ((), ())), preferred_element_type=jnp.float32
    )


def kernel(x, w_mat, scale_x, scale_w):
    m_glob, k_loc = x.shape
    k_glob, n_out = w_mat.shape
    mb = m_glob // N_DEV
    n_chunk = 4
    spc = N_DEV // n_chunk
    do_comm = _ABLATE != "nocomm"

    def body(x_hbm, w_hbm, acc_hbm,
             xf_ref, xb_ref, xg_ref, wf_ref, wg_ref, acc_ref,
             send_sems, recv_sems, wdma_sems, xdma_sems, odma_sem):
        me = lax.axis_index("i")

        def kblk(p):
            return (me - p) % N_DEV

        def start_wdma(step, slot):
            cp = pltpu.make_async_copy(
                w_hbm.at[pl.ds(kblk(step) * k_loc, k_loc), :],
                wf_ref.at[slot],
                wdma_sems.at[slot],
            )
            cp.start()
            return cp

        def start_xdma(i, slot):
            cp = pltpu.make_async_copy(
                x_hbm.at[pl.ds(((me + i + 1) % N_DEV) * mb, mb), :],
                xf_ref.at[slot],
                xdma_sems.at[slot],
            )
            cp.start()
            return cp

        xcp = [start_xdma(0, 0), start_xdma(1, 1)]
        wcp = [start_wdma(0, 0), start_wdma(1, 1)]

        if do_comm:
            barrier = pltpu.get_barrier_semaphore()
            for d in range(1, N_DEV):
                pl.semaphore_signal(
                    barrier, inc=1,
                    device_id=((me + d) % N_DEV,),
                    device_id_type=pl.DeviceIdType.MESH,
                )
            pl.semaphore_wait(barrier, N_DEV - 1)

        rdmas = []
        for i in range(N_DEV):
            d = i + 1
            xcp[i].wait()
            if d < N_DEV:
                xb_ref[d] = xf_ref[i % 2].astype(jnp.float8_e4m3fn)
                if do_comm:
                    rdma = pltpu.make_async_remote_copy(
                        src_ref=xb_ref.at[d],
                        dst_ref=xg_ref.at[:, pl.ds(d * k_loc, k_loc)],
                        send_sem=send_sems.at[d - 1],
                        recv_sem=recv_sems.at[d - 1],
                        device_id=((me + d) % N_DEV,),
                        device_id_type=pl.DeviceIdType.MESH,
                    )
                    rdma.start()
                    rdmas.append(rdma)
            else:
                xg_ref[:, 0:k_loc] = xf_ref[i % 2].astype(jnp.float8_e4m3fn)
            if i + 2 < N_DEV:
                xcp.append(start_xdma(i + 2, i % 2))

        for c in range(n_chunk):
            lo, hi = c * spc, (c + 1) * spc
            for p in range(lo, hi):
                wcp[p].wait()
                wg_ref[p * k_loc:(p + 1) * k_loc, :] = (
                    wf_ref[p % 2].astype(_MM_DTYPE)
                )
                if p + 2 < N_DEV:
                    wcp.append(start_wdma(p + 2, p % 2))
            if do_comm:
                for d in range(max(lo, 1), hi):
                    rdmas[d - 1].wait_recv()
            g = _mm(
                xg_ref[:, lo * k_loc:hi * k_loc],
                wg_ref[lo * k_loc:hi * k_loc, :],
            )
            if c == 0:
                acc_ref[...] = g
            else:
                acc_ref[...] += g

        ocp = pltpu.make_async_copy(acc_ref, acc_hbm, odma_sem)
        ocp.start()
        if do_comm:
            for d in range(1, N_DEV):
                rdmas[d - 1].wait_send()
        ocp.wait()

    acc = pl.pallas_call(
        body,
        out_shape=jax.ShapeDtypeStruct((mb, n_out), jnp.float32),
        in_specs=[
            pl.BlockSpec(memory_space=pltpu.MemorySpace.HBM),
            pl.BlockSpec(memory_space=pltpu.MemorySpace.HBM),
        ],
        out_specs=pl.BlockSpec(memory_space=pltpu.MemorySpace.HBM),
        scratch_shapes=[
            pltpu.VMEM((2, mb, k_loc), jnp.float32),
            pltpu.VMEM((N_DEV, mb, k_loc), jnp.float8_e4m3fn),
            pltpu.VMEM((mb, k_glob), jnp.float8_e4m3fn),
            pltpu.VMEM((2, k_loc, n_out), jnp.float32),
            pltpu.VMEM((k_glob, n_out), _MM_DTYPE),
            pltpu.VMEM((mb, n_out), jnp.float32),
            pltpu.SemaphoreType.DMA((N_DEV - 1,)),
            pltpu.SemaphoreType.DMA((N_DEV - 1,)),
            pltpu.SemaphoreType.DMA((2,)),
            pltpu.SemaphoreType.DMA((2,)),
            pltpu.SemaphoreType.DMA,
        ],
        compiler_params=pltpu.CompilerParams(
            collective_id=None if not do_comm else 0,
            vmem_limit_bytes=100 * 1024 * 1024,
        ),
        interpret=pltpu.InterpretParams() if _INTERPRET else False,
    )(x, w_mat)

    def epilogue(acc_ref, sx_ref, sw_ref, out_ref):
        y = acc_ref[...] * (sx_ref[0] * sw_ref[0])
        out_ref[...] = y * jax.nn.sigmoid(jnp.clip(y, -60.0, 60.0))

    return pl.pallas_call(
        epilogue,
        out_shape=jax.ShapeDtypeStruct((mb, n_out), jnp.float32),
        in_specs=[
            pl.BlockSpec(memory_space=pltpu.VMEM),
            pl.BlockSpec(memory_space=pltpu.SMEM),
            pl.BlockSpec(memory_space=pltpu.SMEM),
        ],
        out_specs=pl.BlockSpec(memory_space=pltpu.VMEM),
        interpret=pltpu.InterpretParams() if _INTERPRET else False,
    )(acc, scale_x, scale_w)
